# baseline (speedup 1.0000x reference)
import functools

import jax
import jax.numpy as jnp
from jax import lax
from jax.experimental import pallas as pl
from jax.experimental.pallas import tpu as pltpu
from jax.experimental.pallas import tpu_sc as plsc

_N = 10000
_NNZ = 320000
_D = 128
_DH = _D // 2
_NC = 2
_NS = 16
_L = 16
_K = 128
_QV = _K // _L
_PS = _NNZ // _NS
_CH = 158
_CHA = _CH + 2
_PSP = _CHA * _K
_RPS = 632
_NP = _RPS * _NS

_SC_PARAMS = pltpu.CompilerParams(use_tc_tiling_on_sc=False,
                                  needs_layout_passes=False)


def _sc_fused_pass():
  mesh = plsc.VectorSubcoreMesh(core_axis_name="c", subcore_axis_name="s")
  out_type = (
      jax.ShapeDtypeStruct((_NC, _NP, _DH), jnp.float32),
      jax.ShapeDtypeStruct((_NS, _NP), jnp.float32),
      jax.ShapeDtypeStruct((_NC, _NP, _DH), jnp.float32),
      jax.ShapeDtypeStruct((_NC, _NS, _NP), jnp.float32),
  )
  scratch = [
      pltpu.VMEM((_CHA, _K), jnp.int32),
      pltpu.VMEM((_CHA, _K), jnp.int32),
      pltpu.VMEM((_K, _DH), jnp.float32),
      pltpu.VMEM((_K, _DH), jnp.float32),
      pltpu.VMEM((_NP,), jnp.float32),
      pltpu.VMEM((_NP,), jnp.float32),
      pltpu.VMEM((_NP,), jnp.float32),
      pltpu.VMEM((_RPS + _L,), jnp.float32),
      pltpu.VMEM((_RPS + _L,), jnp.float32),
      pltpu.VMEM_SHARED((_NP, _DH), jnp.float32),
      pltpu.SemaphoreType.DMA,
      pltpu.SemaphoreType.DMA,
  ]

  @functools.partial(pl.kernel, out_type=out_type, mesh=mesh,
                     scratch_types=scratch, compiler_params=_SC_PARAMS)
  def run(tbl0, tbl1, wvec, zmain, zhist, ng, es, eg, ns,
          out_main, out_d, out_tn, out_bh,
          gv, sv, bufa, bufb, wv, bh, dh, bdeg, tmp,
          acc, sem_a, sem_b):
    c = lax.axis_index("c")
    s = lax.axis_index("s")
    srow = pl.ds(s * _RPS, _RPS)
    pltpu.sync_copy(zmain, acc.at[srow])
    pltpu.sync_copy(ng.at[s], gv)
    pltpu.sync_copy(es.at[s], sv)
    pltpu.sync_copy(wvec, wv)
    pltpu.sync_copy(zhist, bh)
    pltpu.sync_copy(zhist, dh)
    plsc.subcore_barrier()

    def hist_step(j, with_d):
      svj = sv.at[j]
      gvj = gv.at[j]
      ones = jnp.ones((_L,), jnp.float32)
      for q in range(_QV):
        sl = pl.ds(q * _L, _L)
        eid = svj[sl]
        plsc.addupdate_scatter(bh, [eid], ones)
        if with_d:
          wvals = plsc.load_gather(wv, [eid])
          plsc.addupdate_scatter(dh, [gvj[sl]], wvals)

    def stream_loop(gather_from, dest, hist_mode):
      pltpu.async_copy(gather_from.at[gv.at[0]], bufa, sem_a)

      def body(t, carry):
        j0 = 2 * t
        j1 = j0 + 1
        pltpu.async_copy(gather_from.at[gv.at[j1]], bufb, sem_b)
        if hist_mode:
          hist_step(j0, hist_mode == "bd")
        pltpu.make_async_copy(gather_from.at[gv.at[j0]],
                              bufa, sem_a).wait()
        pltpu.sync_copy(bufa, dest.at[sv.at[j0]], add=True)
        pltpu.async_copy(gather_from.at[gv.at[j0 + 2]], bufa, sem_a)
        if hist_mode:
          hist_step(j1, hist_mode == "bd")
        pltpu.make_async_copy(gather_from.at[gv.at[j1]],
                              bufb, sem_b).wait()
        pltpu.sync_copy(bufb, dest.at[sv.at[j1]], add=True)
        return carry

      lax.fori_loop(0, _CH // 2, body, 0)
      pltpu.make_async_copy(gather_from.at[gv.at[0]], bufa, sem_a).wait()

    @pl.when(c == 0)
    def _l1c0():
      stream_loop(tbl0, acc, "b")

    @pl.when(c == 1)
    def _l1c1():
      stream_loop(tbl1, acc, "bd")

    plsc.subcore_barrier()
    pltpu.sync_copy(bh, out_bh.at[c, s])
    pltpu.sync_copy(eg.at[s], gv)
    pltpu.sync_copy(ns.at[s], sv)
    plsc.subcore_barrier()

    _NG = (_RPS + _L - 1) // _L
    pltpu.sync_copy(out_bh.at[c, 0, srow], bdeg.at[pl.ds(0, _RPS)])
    for sp in range(1, _NS):
      pltpu.sync_copy(out_bh.at[c, sp, srow], tmp.at[pl.ds(0, _RPS)])
      for q in range(_NG):
        sl = pl.ds(q * _L, _L)
        bdeg[sl] = bdeg[sl] + tmp[sl]
    for q in range(_NG):
      sl = pl.ds(q * _L, _L)
      bv = bdeg[sl]
      bdeg[sl] = jnp.where(bv > 0, 1.0 / bv, 0.0)

    row0 = s * _RPS
    for blk, rows in ((0, 128), (128, 128), (256, 128), (384, 128),
                      (512, 120)):
      seg = pl.ds(row0 + blk, rows)
      pltpu.sync_copy(acc.at[seg], bufa.at[pl.ds(0, rows)])

      def scale_row(r, carry):
        bsc = bdeg[pl.ds(blk + r, _L)][0]
        for q in range(_DH // _L):
          sl = pl.ds(q * _L, _L)
          bufa[r, sl] = bufa[r, sl] * bsc
        return carry

      lax.fori_loop(0, rows, scale_row, 0)
      pltpu.sync_copy(bufa.at[pl.ds(0, rows)], out_tn.at[c, seg])
    pltpu.sync_copy(zmain, acc.at[srow])
    plsc.subcore_barrier()

    @pl.when(c == 0)
    def _l2c0():
      stream_loop(out_tn.at[0], acc, None)

    @pl.when(c == 1)
    def _l2c1():
      stream_loop(out_tn.at[1], acc, None)

    plsc.subcore_barrier()
    pltpu.sync_copy(acc.at[srow], out_main.at[c, srow])

    @pl.when(c == 1)
    def _wb_d():
      pltpu.sync_copy(dh, out_d.at[s])

  return run


_fused = _sc_fused_pass()

_BLK = 1000
_GRID = _N // _BLK


@functools.partial(
    pl.pallas_call,
    grid=(_GRID,),
    in_specs=[
        pl.BlockSpec((_BLK, _D), lambda i: (i, 0)),
        pl.BlockSpec((_D, _D), lambda i: (0, 0)),
    ],
    out_specs=[
        pl.BlockSpec((_BLK, _DH), lambda i: (i, 0)),
        pl.BlockSpec((_BLK, _DH), lambda i: (i, 0)),
    ],
    out_shape=[
        jax.ShapeDtypeStruct((_N, _DH), jnp.float32),
        jax.ShapeDtypeStruct((_N, _DH), jnp.float32),
    ],
)
def _matmul(x_ref, w_ref, o0_ref, o1_ref):
  xl = jnp.dot(x_ref[...], w_ref[...], preferred_element_type=jnp.float32)
  o0_ref[...] = xl[:, :_DH]
  o1_ref[...] = xl[:, _DH:]


@functools.partial(
    pl.pallas_call,
    out_shape=jax.ShapeDtypeStruct((_N, _D), jnp.float32),
)
def _combine_out(pn_ref, pd_ref, b_ref, o_ref):
  deg = jnp.sum(pd_ref[...], axis=0)[:_N, None]
  dinv = jnp.where(deg > 0, 1.0 / deg, 0.0)
  full = jnp.concatenate(
      [dinv * pn_ref[0, :_N, :], dinv * pn_ref[1, :_N, :]], axis=1)
  o_ref[...] = jnp.maximum(full + b_ref[...], 0.0)


def _pad_idx(idx, fill):
  per_s = idx.reshape(_NS, _PS)
  padded = jnp.pad(per_s, ((0, 0), (0, _PSP - _PS)), constant_values=fill)
  return padded.reshape(_NS, _CHA, _K)


def kernel(x, hyperedge_index, hyperedge_weight, W, b):
  x = x.astype(jnp.float32)
  node_idx = hyperedge_index[0].astype(jnp.int32)
  edge_idx = hyperedge_index[1].astype(jnp.int32)
  node_g = _pad_idx(node_idx, 0)
  node_s = _pad_idx(node_idx, _N)
  edge_g = _pad_idx(edge_idx, 0)
  edge_s = _pad_idx(edge_idx, _N)

  xl0, xl1 = _matmul(x, W.astype(jnp.float32))

  zmain = jnp.zeros((_RPS, _DH), jnp.float32)
  zhist = jnp.zeros((_NP,), jnp.float32)
  wvec = jnp.pad(hyperedge_weight.astype(jnp.float32), (0, _NP - _N))

  pn, pd, _, _ = _fused(xl0, xl1, wvec, zmain, zhist,
                        node_g, edge_s, edge_g, node_s)
  return _combine_out(pn, pd, b.astype(jnp.float32).reshape(1, _D))
